# baseline (speedup 1.0000x reference)
import functools

import jax
import jax.numpy as jnp
from jax import lax
from jax.experimental import pallas as pl
from jax.experimental.pallas import tpu as pltpu
from jax.experimental.pallas import tpu_sc as plsc

NC = 2
NS = 16
NW = NC * NS
L = 16


def _sc_mesh():
    return plsc.VectorSubcoreMesh(core_axis_name="c", subcore_axis_name="s",
                                  num_cores=NC, num_subcores=NS)


def _sc_params():
    return pltpu.CompilerParams(needs_layout_passes=False)


def _make_degree_kernel(E, NPAD):
    CE = E // NW

    @functools.partial(
        pl.kernel,
        out_type=jax.ShapeDtypeStruct((NW, 1, 2 * NPAD), jnp.float32),
        mesh=_sc_mesh(),
        compiler_params=_sc_params(),
        scratch_types=[
            pltpu.VMEM((2, CE), jnp.int32),
            pltpu.VMEM((2 * NPAD,), jnp.float32),
        ],
    )
    def deg_k(edges, out, idx_v, hist_v):
        c = lax.axis_index("c")
        s = lax.axis_index("s")
        wid = c * NS + s

        zeros = jnp.zeros((L,), jnp.float32)

        def zero_body(i, _):
            hist_v[pl.ds(i * L, L)] = zeros
            return _

        lax.fori_loop(0, 2 * NPAD // L, zero_body, None)

        pltpu.sync_copy(edges.at[0, wid], idx_v.at[0])
        pltpu.sync_copy(edges.at[1, wid], idx_v.at[1])

        ones = jnp.ones((L,), jnp.float32)
        off = jnp.full((L,), NPAD, jnp.int32)

        def edge_body(i, _):
            sv = idx_v[0, pl.ds(i * L, L)]
            dv = idx_v[1, pl.ds(i * L, L)]
            plsc.addupdate_scatter(hist_v, [sv], ones)
            plsc.addupdate_scatter(hist_v, [dv + off], ones)
            return _

        lax.fori_loop(0, CE // L, edge_body, None)

        pltpu.sync_copy(hist_v, out.at[wid, 0])

    return deg_k


def _make_agg_kernel(NP, E, D, KB, CHB):
    CE = E // NW
    NB = CE // KB
    NCH = NB // CHB
    assert NB % CHB == 0
    NR = NP // NS
    assert NR % KB == 0 and NR % 8 == 0 and KB % 8 == 0

    @functools.partial(
        pl.kernel,
        out_type=jax.ShapeDtypeStruct((NC, NP, D), jnp.float32),
        mesh=_sc_mesh(),
        compiler_params=_sc_params(),
        scratch_types=[
            pltpu.VMEM((CHB, KB), jnp.int32),
            pltpu.VMEM((CHB, KB), jnp.int32),
            pltpu.VMEM((KB, D), jnp.float32),
            pltpu.VMEM((KB, D), jnp.float32),
            pltpu.VMEM((KB, D), jnp.float32),
            pltpu.VMEM_SHARED((NP, D), jnp.float32),
        ] + [pltpu.SemaphoreType.DMA] * 6,
    )
    def agg_k(edges, table, out, src_v, dst_v, rows0_v, rows1_v, rows2_v,
              agg_sh, gsem0, gsem1, gsem2, ssem0, ssem1, ssem2):
        c = lax.axis_index("c")
        s = lax.axis_index("s")
        wid = c * NS + s

        zeros = jnp.zeros((L,), jnp.float32)
        DL = D // L

        def zero_body(i, _):
            rows0_v[i // DL, pl.ds((i % DL) * L, L)] = zeros
            return _

        lax.fori_loop(0, KB * DL, zero_body, None)

        def zero_dma(k, _):
            pltpu.sync_copy(rows0_v, agg_sh.at[pl.ds(s * NR + k * KB, KB)])
            return _

        lax.fori_loop(0, NR // KB, zero_dma, None)
        plsc.subcore_barrier()

        R = 3
        bufs = (rows0_v, rows1_v, rows2_v)
        gsems = (gsem0, gsem1, gsem2)
        ssems = (ssem0, ssem1, ssem2)

        def gather(j, p):
            return pltpu.make_async_copy(table.at[src_v.at[j]], bufs[p],
                                         gsems[p])

        def scat_wait(j, p):
            return pltpu.make_async_copy(bufs[p], agg_sh.at[dst_v.at[j]],
                                         ssems[p])

        def chunk_body(ch, _):
            pltpu.sync_copy(edges.at[0, wid, ch], src_v)
            pltpu.sync_copy(edges.at[1, wid, ch], dst_v)
            for k in range(R - 1):
                gather(k, k).start()

            def blk_body(j, _):
                for p in range(R):
                    @pl.when(j % R == p)
                    def _():
                        gather(j, p).wait()
                        pltpu.async_copy(bufs[p], agg_sh.at[dst_v.at[j]],
                                         ssems[p], add=True)

                        @pl.when(j + R - 1 < CHB)
                        def _():
                            q = (p + R - 1) % R
                            @pl.when(j > 0)
                            def _():
                                scat_wait(j - 1, q).wait()
                            gather(j + R - 1, q).start()
                return _

            lax.fori_loop(0, CHB, blk_body, None)
            for t in range(CHB - R, CHB):
                scat_wait(t, t % R).wait()
            return _

        lax.fori_loop(0, NCH, chunk_body, None)
        plsc.subcore_barrier()

        pltpu.sync_copy(agg_sh.at[pl.ds(s * NR, NR)],
                        out.at[c, pl.ds(s * NR, NR)])

    return agg_k


def _make_agg_cols_kernel(NP, E, DO, CED):
    EC = E // NC
    NCH = EC // CED
    assert EC % CED == 0 and DO == NS

    @functools.partial(
        pl.kernel,
        out_type=jax.ShapeDtypeStruct((NC, DO, 1, NP), jnp.float32),
        mesh=_sc_mesh(),
        compiler_params=_sc_params(),
        scratch_types=[
            pltpu.VMEM((NP,), jnp.float32),
            pltpu.VMEM((NP,), jnp.float32),
            pltpu.VMEM((2, CED), jnp.int32),
            pltpu.VMEM((2, CED), jnp.int32),
            pltpu.SemaphoreType.DMA,
            pltpu.SemaphoreType.DMA,
        ],
    )
    def aggc_k(tableT, edges, out, col_v, acc_v, idxA_v, idxB_v, semA, semB):
        c = lax.axis_index("c")
        s = lax.axis_index("s")

        zeros = jnp.zeros((L,), jnp.float32)

        def zero_body(i, _):
            acc_v[pl.ds(i * L, L)] = zeros
            return _

        lax.fori_loop(0, NP // L, zero_body, None)

        pltpu.sync_copy(tableT.at[s, 0], col_v)

        ibufs = (idxA_v, idxB_v)
        isems = (semA, semB)

        def idx_load(ch, p):
            def one(r):
                return pltpu.make_async_copy(
                    edges.at[r, c, ch, 0], ibufs[p].at[r], isems[p])
            return one

        idx_load(0, 0)(0).start()
        idx_load(0, 0)(1).start()

        def chunk_body(ch, _):
            for p in range(2):
                @pl.when(ch % 2 == p)
                def _():
                    idx_load(ch, p)(0).wait()
                    idx_load(ch, p)(1).wait()

                    @pl.when(ch < NCH - 1)
                    def _():
                        idx_load(ch + 1, 1 - p)(0).start()
                        idx_load(ch + 1, 1 - p)(1).start()

                    U = 5

                    def edge_body(i, _):
                        for u in range(U):
                            k = i * U + u
                            sv = ibufs[p][0, pl.ds(k * L, L)]
                            dv = ibufs[p][1, pl.ds(k * L, L)]
                            vals = plsc.load_gather(col_v, [sv])
                            plsc.addupdate_scatter(acc_v, [dv], vals)
                        return _

                    lax.fori_loop(0, CED // (L * U), edge_body, None)
            return _

        lax.fori_loop(0, NCH, chunk_body, None)

        pltpu.sync_copy(acc_v, out.at[c, s, 0])

    return aggc_k


def _make_tc_layer1(N, NPAD, DIN, DH, BN):
    def body(hist_ref, feat_ref, w_ref, out_ref):
        h = hist_ref[...]
        deg_out = jnp.sum(h[:, :NW], axis=1, keepdims=True)
        ns = lax.rsqrt(jnp.maximum(deg_out, 1.0))
        x = feat_ref[...] * ns
        out_ref[...] = jnp.dot(x, w_ref[...], preferred_element_type=jnp.float32)

    return pl.pallas_call(
        body,
        grid=(NPAD // BN,),
        in_specs=[
            pl.BlockSpec((BN, 2 * NW), lambda i: (i, 0)),
            pl.BlockSpec((BN, DIN), lambda i: (i, 0)),
            pl.BlockSpec((DIN, DH), lambda i: (0, 0)),
        ],
        out_specs=pl.BlockSpec((BN, DH), lambda i: (i, 0)),
        out_shape=jax.ShapeDtypeStruct((N, DH), jnp.float32),
    )


def _make_tc_mid(N, NPAD, DH, DO, BN):
    def body(hist_ref, aggp_ref, b1_ref, w2_ref, out_ref):
        h = hist_ref[...]
        deg_out = jnp.sum(h[:, :NW], axis=1, keepdims=True)
        deg_in = jnp.sum(h[:, NW:], axis=1, keepdims=True)
        ns = lax.rsqrt(jnp.maximum(deg_out, 1.0))
        nd = lax.rsqrt(jnp.maximum(deg_in, 1.0))
        agg = aggp_ref[0] + aggp_ref[1]
        out1 = jnp.maximum(agg * nd + b1_ref[...], 0.0)
        out_ref[...] = jnp.dot(out1, w2_ref[...],
                               preferred_element_type=jnp.float32) * ns

    return pl.pallas_call(
        body,
        grid=(NPAD // BN,),
        in_specs=[
            pl.BlockSpec((BN, 2 * NW), lambda i: (i, 0)),
            pl.BlockSpec((NC, BN, DH), lambda i: (0, i, 0)),
            pl.BlockSpec((1, DH), lambda i: (0, 0)),
            pl.BlockSpec((DH, DO), lambda i: (0, 0)),
        ],
        out_specs=pl.BlockSpec((BN, DO), lambda i: (i, 0)),
        out_shape=jax.ShapeDtypeStruct((NPAD, DO), jnp.float32),
    )


def _make_tc_final(N, NPAD, DO, BN):
    def body(hist_ref, aggp_ref, b2_ref, out_ref):
        h = hist_ref[...]
        deg_in = jnp.sum(h[:, NW:], axis=1, keepdims=True)
        nd = lax.rsqrt(jnp.maximum(deg_in, 1.0))
        agg = aggp_ref[0] + aggp_ref[1]
        out_ref[...] = agg * nd + b2_ref[...]

    return pl.pallas_call(
        body,
        grid=(NPAD // BN,),
        in_specs=[
            pl.BlockSpec((BN, 2 * NW), lambda i: (i, 0)),
            pl.BlockSpec((NC, BN, DO), lambda i: (0, i, 0)),
            pl.BlockSpec((1, DO), lambda i: (0, 0)),
        ],
        out_specs=pl.BlockSpec((BN, DO), lambda i: (i, 0)),
        out_shape=jax.ShapeDtypeStruct((N, DO), jnp.float32),
    )


def kernel(features, edge_index, W1, b1, W2, b2):
    N, DIN = features.shape
    E = edge_index.shape[1]
    DH = W1.shape[1]
    DO = W2.shape[1]

    KB = 80
    CHB = 25
    CED = 10000
    assert E % (NW * KB * CHB) == 0 and N % NS == 0 and E % (NC * CED) == 0
    CE = E // NW
    NB = CE // KB
    BN = 1024
    NPAD = -(-N // BN) * BN

    edges_deg = edge_index.reshape(2, NW, CE)
    edges_blk = edge_index.reshape(2, NW, NB // CHB, CHB, KB)
    edges_half = edge_index.reshape(2, NC, E // (NC * CED), 1, CED)

    histp = _make_degree_kernel(E, NPAD)(edges_deg)
    hist_t = (histp.reshape(NW, 2, NPAD)
              .transpose(2, 1, 0).reshape(NPAD, 2 * NW))

    h1s = _make_tc_layer1(N, NPAD, DIN, DH, BN)(hist_t, features, W1)
    aggp1 = _make_agg_kernel(NPAD, E, DH, KB, CHB)(edges_blk, h1s)
    h2s = _make_tc_mid(N, NPAD, DH, DO, BN)(
        hist_t, aggp1, b1.reshape(1, DH), W2)

    h2sT = h2s.T.reshape(DO, 1, NPAD)
    aggp2 = _make_agg_cols_kernel(NPAD, E, DO, CED)(h2sT, edges_half)
    aggp2 = aggp2.reshape(NC, DO, NPAD).transpose(0, 2, 1)

    out = _make_tc_final(N, NPAD, DO, BN)(
        hist_t, aggp2, b2.reshape(1, DO))
    return out


# speedup vs baseline: 1.2695x; 1.2695x over previous
import functools

import jax
import jax.numpy as jnp
from jax import lax
from jax.experimental import pallas as pl
from jax.experimental.pallas import tpu as pltpu
from jax.experimental.pallas import tpu_sc as plsc

NC = 2
NS = 16
NW = NC * NS
L = 16


def _sc_mesh():
    return plsc.VectorSubcoreMesh(core_axis_name="c", subcore_axis_name="s",
                                  num_cores=NC, num_subcores=NS)


def _sc_params():
    return pltpu.CompilerParams(needs_layout_passes=False)


def _make_degree_kernel(E, NPAD):
    CE = E // NW

    @functools.partial(
        pl.kernel,
        out_type=jax.ShapeDtypeStruct((NW, 1, 2 * NPAD), jnp.float32),
        mesh=_sc_mesh(),
        compiler_params=_sc_params(),
        scratch_types=[
            pltpu.VMEM((2, CE), jnp.int32),
            pltpu.VMEM((2 * NPAD,), jnp.float32),
        ],
    )
    def deg_k(edges, out, idx_v, hist_v):
        c = lax.axis_index("c")
        s = lax.axis_index("s")
        wid = c * NS + s

        zeros = jnp.zeros((L,), jnp.float32)

        def zero_body(i, _):
            hist_v[pl.ds(i * L, L)] = zeros
            return _

        lax.fori_loop(0, 2 * NPAD // L, zero_body, None)

        pltpu.sync_copy(edges.at[0, wid], idx_v.at[0])
        pltpu.sync_copy(edges.at[1, wid], idx_v.at[1])

        ones = jnp.ones((L,), jnp.float32)
        off = jnp.full((L,), NPAD, jnp.int32)

        @plsc.parallel_loop(0, CE // L, unroll=8)
        def _(i):
            sv = idx_v[0, pl.ds(i * L, L)]
            dv = idx_v[1, pl.ds(i * L, L)]
            plsc.addupdate_scatter(hist_v, [sv], ones)
            plsc.addupdate_scatter(hist_v, [dv + off], ones)

        pltpu.sync_copy(hist_v, out.at[wid, 0])

    return deg_k


def _make_agg_kernel(NP, E, D, KB, CHB):
    CE = E // NW
    NB = CE // KB
    NCH = NB // CHB
    assert NB % CHB == 0
    NR = NP // NS
    assert NR % KB == 0 and NR % 8 == 0 and KB % 8 == 0

    @functools.partial(
        pl.kernel,
        out_type=jax.ShapeDtypeStruct((NC, NP, D), jnp.float32),
        mesh=_sc_mesh(),
        compiler_params=_sc_params(),
        scratch_types=[
            pltpu.VMEM((CHB, KB), jnp.int32),
            pltpu.VMEM((CHB, KB), jnp.int32),
            pltpu.VMEM((KB, D), jnp.float32),
            pltpu.VMEM((KB, D), jnp.float32),
            pltpu.VMEM((KB, D), jnp.float32),
            pltpu.VMEM_SHARED((NP, D), jnp.float32),
        ] + [pltpu.SemaphoreType.DMA] * 6,
    )
    def agg_k(edges, table, out, src_v, dst_v, rows0_v, rows1_v, rows2_v,
              agg_sh, gsem0, gsem1, gsem2, ssem0, ssem1, ssem2):
        c = lax.axis_index("c")
        s = lax.axis_index("s")
        wid = c * NS + s

        zeros = jnp.zeros((L,), jnp.float32)
        DL = D // L

        def zero_body(i, _):
            rows0_v[i // DL, pl.ds((i % DL) * L, L)] = zeros
            return _

        lax.fori_loop(0, KB * DL, zero_body, None)

        def zero_dma(k, _):
            pltpu.sync_copy(rows0_v, agg_sh.at[pl.ds(s * NR + k * KB, KB)])
            return _

        lax.fori_loop(0, NR // KB, zero_dma, None)
        plsc.subcore_barrier()

        R = 3
        bufs = (rows0_v, rows1_v, rows2_v)
        gsems = (gsem0, gsem1, gsem2)
        ssems = (ssem0, ssem1, ssem2)

        def gather(j, p):
            return pltpu.make_async_copy(table.at[src_v.at[j]], bufs[p],
                                         gsems[p])

        def scat_wait(j, p):
            return pltpu.make_async_copy(bufs[p], agg_sh.at[dst_v.at[j]],
                                         ssems[p])

        def chunk_body(ch, _):
            pltpu.sync_copy(edges.at[0, wid, ch], src_v)
            pltpu.sync_copy(edges.at[1, wid, ch], dst_v)
            for k in range(R - 1):
                gather(k, k).start()

            def blk_body(j, _):
                for p in range(R):
                    @pl.when(j % R == p)
                    def _():
                        gather(j, p).wait()
                        pltpu.async_copy(bufs[p], agg_sh.at[dst_v.at[j]],
                                         ssems[p], add=True)

                        @pl.when(j + R - 1 < CHB)
                        def _():
                            q = (p + R - 1) % R
                            @pl.when(j > 0)
                            def _():
                                scat_wait(j - 1, q).wait()
                            gather(j + R - 1, q).start()
                return _

            lax.fori_loop(0, CHB, blk_body, None)
            for t in range(CHB - R, CHB):
                scat_wait(t, t % R).wait()
            return _

        lax.fori_loop(0, NCH, chunk_body, None)
        plsc.subcore_barrier()

        pltpu.sync_copy(agg_sh.at[pl.ds(s * NR, NR)],
                        out.at[c, pl.ds(s * NR, NR)])

    return agg_k


def _make_agg_cols_kernel(NP, E, DO, CED):
    EC = E // NC
    NCH = EC // CED
    assert EC % CED == 0 and DO == NS

    @functools.partial(
        pl.kernel,
        out_type=jax.ShapeDtypeStruct((NC, DO, 1, NP), jnp.float32),
        mesh=_sc_mesh(),
        compiler_params=_sc_params(),
        scratch_types=[
            pltpu.VMEM((NP,), jnp.float32),
            pltpu.VMEM((NP,), jnp.float32),
            pltpu.VMEM((2, CED), jnp.int32),
            pltpu.VMEM((2, CED), jnp.int32),
            pltpu.SemaphoreType.DMA,
            pltpu.SemaphoreType.DMA,
        ],
    )
    def aggc_k(tableT, edges, out, col_v, acc_v, idxA_v, idxB_v, semA, semB):
        c = lax.axis_index("c")
        s = lax.axis_index("s")

        zeros = jnp.zeros((L,), jnp.float32)

        def zero_body(i, _):
            acc_v[pl.ds(i * L, L)] = zeros
            return _

        lax.fori_loop(0, NP // L, zero_body, None)

        pltpu.sync_copy(tableT.at[s, 0], col_v)

        ibufs = (idxA_v, idxB_v)
        isems = (semA, semB)

        def idx_load(ch, p):
            def one(r):
                return pltpu.make_async_copy(
                    edges.at[r, c, ch, 0], ibufs[p].at[r], isems[p])
            return one

        idx_load(0, 0)(0).start()
        idx_load(0, 0)(1).start()

        def chunk_body(ch, _):
            for p in range(2):
                @pl.when(ch % 2 == p)
                def _():
                    idx_load(ch, p)(0).wait()
                    idx_load(ch, p)(1).wait()

                    @pl.when(ch < NCH - 1)
                    def _():
                        idx_load(ch + 1, 1 - p)(0).start()
                        idx_load(ch + 1, 1 - p)(1).start()

                    @plsc.parallel_loop(0, CED // L, unroll=8)
                    def _(k):
                        sv = ibufs[p][0, pl.ds(k * L, L)]
                        dv = ibufs[p][1, pl.ds(k * L, L)]
                        vals = plsc.load_gather(col_v, [sv])
                        plsc.addupdate_scatter(acc_v, [dv], vals)
            return _

        lax.fori_loop(0, NCH, chunk_body, None)

        pltpu.sync_copy(acc_v, out.at[c, s, 0])

    return aggc_k


def _make_tc_layer1(N, NPAD, DIN, DH, BN):
    def body(hist_ref, feat_ref, w_ref, out_ref):
        h = hist_ref[...]
        deg_out = jnp.sum(h[:, :NW], axis=1, keepdims=True)
        ns = lax.rsqrt(jnp.maximum(deg_out, 1.0))
        x = feat_ref[...] * ns
        out_ref[...] = jnp.dot(x, w_ref[...], preferred_element_type=jnp.float32)

    return pl.pallas_call(
        body,
        grid=(NPAD // BN,),
        in_specs=[
            pl.BlockSpec((BN, 2 * NW), lambda i: (i, 0)),
            pl.BlockSpec((BN, DIN), lambda i: (i, 0)),
            pl.BlockSpec((DIN, DH), lambda i: (0, 0)),
        ],
        out_specs=pl.BlockSpec((BN, DH), lambda i: (i, 0)),
        out_shape=jax.ShapeDtypeStruct((N, DH), jnp.float32),
    )


def _make_tc_mid(N, NPAD, DH, DO, BN):
    def body(hist_ref, aggp_ref, b1_ref, w2_ref, out_ref):
        h = hist_ref[...]
        deg_out = jnp.sum(h[:, :NW], axis=1, keepdims=True)
        deg_in = jnp.sum(h[:, NW:], axis=1, keepdims=True)
        ns = lax.rsqrt(jnp.maximum(deg_out, 1.0))
        nd = lax.rsqrt(jnp.maximum(deg_in, 1.0))
        agg = aggp_ref[0] + aggp_ref[1]
        out1 = jnp.maximum(agg * nd + b1_ref[...], 0.0)
        out_ref[...] = jnp.dot(out1, w2_ref[...],
                               preferred_element_type=jnp.float32) * ns

    return pl.pallas_call(
        body,
        grid=(NPAD // BN,),
        in_specs=[
            pl.BlockSpec((BN, 2 * NW), lambda i: (i, 0)),
            pl.BlockSpec((NC, BN, DH), lambda i: (0, i, 0)),
            pl.BlockSpec((1, DH), lambda i: (0, 0)),
            pl.BlockSpec((DH, DO), lambda i: (0, 0)),
        ],
        out_specs=pl.BlockSpec((BN, DO), lambda i: (i, 0)),
        out_shape=jax.ShapeDtypeStruct((NPAD, DO), jnp.float32),
    )


def _make_tc_final(N, NPAD, DO, BN):
    def body(hist_ref, aggp_ref, b2_ref, out_ref):
        h = hist_ref[...]
        deg_in = jnp.sum(h[:, NW:], axis=1, keepdims=True)
        nd = lax.rsqrt(jnp.maximum(deg_in, 1.0))
        agg = aggp_ref[0] + aggp_ref[1]
        out_ref[...] = agg * nd + b2_ref[...]

    return pl.pallas_call(
        body,
        grid=(NPAD // BN,),
        in_specs=[
            pl.BlockSpec((BN, 2 * NW), lambda i: (i, 0)),
            pl.BlockSpec((NC, BN, DO), lambda i: (0, i, 0)),
            pl.BlockSpec((1, DO), lambda i: (0, 0)),
        ],
        out_specs=pl.BlockSpec((BN, DO), lambda i: (i, 0)),
        out_shape=jax.ShapeDtypeStruct((N, DO), jnp.float32),
    )


def kernel(features, edge_index, W1, b1, W2, b2):
    N, DIN = features.shape
    E = edge_index.shape[1]
    DH = W1.shape[1]
    DO = W2.shape[1]

    KB = 80
    CHB = 25
    CED = 10000
    assert E % (NW * KB * CHB) == 0 and N % NS == 0 and E % (NC * CED) == 0
    CE = E // NW
    NB = CE // KB
    BN = 1024
    NPAD = -(-N // BN) * BN

    edges_deg = edge_index.reshape(2, NW, CE)
    edges_blk = edge_index.reshape(2, NW, NB // CHB, CHB, KB)
    edges_half = edge_index.reshape(2, NC, E // (NC * CED), 1, CED)

    histp = _make_degree_kernel(E, NPAD)(edges_deg)
    hist_t = (histp.reshape(NW, 2, NPAD)
              .transpose(2, 1, 0).reshape(NPAD, 2 * NW))

    h1s = _make_tc_layer1(N, NPAD, DIN, DH, BN)(hist_t, features, W1)
    aggp1 = _make_agg_kernel(NPAD, E, DH, KB, CHB)(edges_blk, h1s)
    h2s = _make_tc_mid(N, NPAD, DH, DO, BN)(
        hist_t, aggp1, b1.reshape(1, DH), W2)

    h2sT = h2s.T.reshape(DO, 1, NPAD)
    aggp2 = _make_agg_cols_kernel(NPAD, E, DO, CED)(h2sT, edges_half)
    aggp2 = aggp2.reshape(NC, DO, NPAD).transpose(0, 2, 1)

    out = _make_tc_final(N, NPAD, DO, BN)(
        hist_t, aggp2, b2.reshape(1, DO))
    return out


# speedup vs baseline: 1.3785x; 1.0859x over previous
import functools

import jax
import jax.numpy as jnp
from jax import lax
from jax.experimental import pallas as pl
from jax.experimental.pallas import tpu as pltpu
from jax.experimental.pallas import tpu_sc as plsc

NC = 2
NS = 16
NW = NC * NS
L = 16


def _sc_mesh():
    return plsc.VectorSubcoreMesh(core_axis_name="c", subcore_axis_name="s",
                                  num_cores=NC, num_subcores=NS)


def _sc_params():
    return pltpu.CompilerParams(needs_layout_passes=False)


def _make_degree_kernel(E, NPAD):
    CE = E // NW
    CW = 2 * NPAD // NS

    @functools.partial(
        pl.kernel,
        out_type=jax.ShapeDtypeStruct((NC, 1, 2 * NPAD), jnp.float32),
        mesh=_sc_mesh(),
        compiler_params=_sc_params(),
        scratch_types=[
            pltpu.VMEM((2, CE), jnp.int32),
            pltpu.VMEM((2 * NPAD,), jnp.float32),
            pltpu.VMEM((CW,), jnp.float32),
            pltpu.VMEM((CW,), jnp.float32),
            pltpu.VMEM_SHARED((NS, 2 * NPAD), jnp.float32),
        ],
    )
    def deg_k(edges, out, idx_v, hist_v, acc_v, tmp_v, hist_sh):
        c = lax.axis_index("c")
        s = lax.axis_index("s")
        wid = c * NS + s

        zeros = jnp.zeros((L,), jnp.float32)

        def zero_body(i, _):
            hist_v[pl.ds(i * L, L)] = zeros
            return _

        lax.fori_loop(0, 2 * NPAD // L, zero_body, None)

        pltpu.sync_copy(edges.at[0, wid], idx_v.at[0])
        pltpu.sync_copy(edges.at[1, wid], idx_v.at[1])

        ones = jnp.ones((L,), jnp.float32)
        off = jnp.full((L,), NPAD, jnp.int32)

        @plsc.parallel_loop(0, CE // L, unroll=8)
        def _(i):
            sv = idx_v[0, pl.ds(i * L, L)]
            dv = idx_v[1, pl.ds(i * L, L)]
            plsc.addupdate_scatter(hist_v, [sv], ones)
            plsc.addupdate_scatter(hist_v, [dv + off], ones)

        pltpu.sync_copy(hist_v, hist_sh.at[s])
        plsc.subcore_barrier()

        def zacc(i, _):
            acc_v[pl.ds(i * L, L)] = zeros
            return _

        lax.fori_loop(0, CW // L, zacc, None)

        def red_tile(t, _):
            pltpu.sync_copy(hist_sh.at[t, pl.ds(s * CW, CW)], tmp_v)

            @plsc.parallel_loop(0, CW // L, unroll=8)
            def _(i):
                acc_v[pl.ds(i * L, L)] += tmp_v[pl.ds(i * L, L)]
            return _

        lax.fori_loop(0, NS, red_tile, None)
        pltpu.sync_copy(acc_v, out.at[c, 0, pl.ds(s * CW, CW)])

    return deg_k


def _make_agg_kernel(NP, E, D, KB, CHB):
    CE = E // NW
    NB = CE // KB
    NCH = NB // CHB
    assert NB % CHB == 0
    NR = NP // NS
    assert NR % KB == 0 and NR % 8 == 0 and KB % 8 == 0

    @functools.partial(
        pl.kernel,
        out_type=jax.ShapeDtypeStruct((NC, NP, D), jnp.float32),
        mesh=_sc_mesh(),
        compiler_params=_sc_params(),
        scratch_types=[
            pltpu.VMEM((CHB, KB), jnp.int32),
            pltpu.VMEM((CHB, KB), jnp.int32),
            pltpu.VMEM((KB, D), jnp.float32),
            pltpu.VMEM((KB, D), jnp.float32),
            pltpu.VMEM((KB, D), jnp.float32),
            pltpu.VMEM_SHARED((NP, D), jnp.float32),
        ] + [pltpu.SemaphoreType.DMA] * 6,
    )
    def agg_k(edges, table, out, src_v, dst_v, rows0_v, rows1_v, rows2_v,
              agg_sh, gsem0, gsem1, gsem2, ssem0, ssem1, ssem2):
        c = lax.axis_index("c")
        s = lax.axis_index("s")
        wid = c * NS + s

        zeros = jnp.zeros((L,), jnp.float32)
        DL = D // L

        def zero_body(i, _):
            rows0_v[i // DL, pl.ds((i % DL) * L, L)] = zeros
            return _

        lax.fori_loop(0, KB * DL, zero_body, None)

        def zero_dma(k, _):
            pltpu.sync_copy(rows0_v, agg_sh.at[pl.ds(s * NR + k * KB, KB)])
            return _

        lax.fori_loop(0, NR // KB, zero_dma, None)
        plsc.subcore_barrier()

        R = 3
        bufs = (rows0_v, rows1_v, rows2_v)
        gsems = (gsem0, gsem1, gsem2)
        ssems = (ssem0, ssem1, ssem2)

        def gather(j, p):
            return pltpu.make_async_copy(table.at[src_v.at[j]], bufs[p],
                                         gsems[p])

        def scat_wait(j, p):
            return pltpu.make_async_copy(bufs[p], agg_sh.at[dst_v.at[j]],
                                         ssems[p])

        def chunk_body(ch, _):
            pltpu.sync_copy(edges.at[0, wid, ch], src_v)
            pltpu.sync_copy(edges.at[1, wid, ch], dst_v)
            for k in range(R - 1):
                gather(k, k).start()

            def blk_body(j, _):
                for p in range(R):
                    @pl.when(j % R == p)
                    def _():
                        gather(j, p).wait()
                        pltpu.async_copy(bufs[p], agg_sh.at[dst_v.at[j]],
                                         ssems[p], add=True)

                        @pl.when(j + R - 1 < CHB)
                        def _():
                            q = (p + R - 1) % R
                            @pl.when(j > 0)
                            def _():
                                scat_wait(j - 1, q).wait()
                            gather(j + R - 1, q).start()
                return _

            lax.fori_loop(0, CHB, blk_body, None)
            for t in range(CHB - R, CHB):
                scat_wait(t, t % R).wait()
            return _

        lax.fori_loop(0, NCH, chunk_body, None)
        plsc.subcore_barrier()

        pltpu.sync_copy(agg_sh.at[pl.ds(s * NR, NR)],
                        out.at[c, pl.ds(s * NR, NR)])

    return agg_k


def _make_agg_cols_kernel(NP, E, DO, CED):
    EC = E // NC
    NCH = EC // CED
    assert EC % CED == 0 and DO == NS

    @functools.partial(
        pl.kernel,
        out_type=jax.ShapeDtypeStruct((NC, DO, 1, NP), jnp.float32),
        mesh=_sc_mesh(),
        compiler_params=_sc_params(),
        scratch_types=[
            pltpu.VMEM((NP,), jnp.float32),
            pltpu.VMEM((NP,), jnp.float32),
            pltpu.VMEM((2, CED), jnp.int32),
            pltpu.VMEM((2, CED), jnp.int32),
            pltpu.SemaphoreType.DMA,
            pltpu.SemaphoreType.DMA,
        ],
    )
    def aggc_k(tableT, edges, out, col_v, acc_v, idxA_v, idxB_v, semA, semB):
        c = lax.axis_index("c")
        s = lax.axis_index("s")

        zeros = jnp.zeros((L,), jnp.float32)

        def zero_body(i, _):
            acc_v[pl.ds(i * L, L)] = zeros
            return _

        lax.fori_loop(0, NP // L, zero_body, None)

        pltpu.sync_copy(tableT.at[s, 0], col_v)

        ibufs = (idxA_v, idxB_v)
        isems = (semA, semB)

        def idx_load(ch, p):
            def one(r):
                return pltpu.make_async_copy(
                    edges.at[r, c, ch, 0], ibufs[p].at[r], isems[p])
            return one

        idx_load(0, 0)(0).start()
        idx_load(0, 0)(1).start()

        def chunk_body(ch, _):
            for p in range(2):
                @pl.when(ch % 2 == p)
                def _():
                    idx_load(ch, p)(0).wait()
                    idx_load(ch, p)(1).wait()

                    @pl.when(ch < NCH - 1)
                    def _():
                        idx_load(ch + 1, 1 - p)(0).start()
                        idx_load(ch + 1, 1 - p)(1).start()

                    @plsc.parallel_loop(0, CED // L, unroll=8)
                    def _(k):
                        sv = ibufs[p][0, pl.ds(k * L, L)]
                        dv = ibufs[p][1, pl.ds(k * L, L)]
                        vals = plsc.load_gather(col_v, [sv])
                        plsc.addupdate_scatter(acc_v, [dv], vals)
            return _

        lax.fori_loop(0, NCH, chunk_body, None)

        pltpu.sync_copy(acc_v, out.at[c, s, 0])

    return aggc_k


def _make_tc_layer1(N, NPAD, DIN, DH, BN):
    def body(hist_ref, feat_ref, w_ref, out_ref):
        h = hist_ref[...]
        deg_out = jnp.sum(h[:, :NC], axis=1, keepdims=True)
        ns = lax.rsqrt(jnp.maximum(deg_out, 1.0))
        x = feat_ref[...] * ns
        out_ref[...] = jnp.dot(x, w_ref[...], preferred_element_type=jnp.float32)

    return pl.pallas_call(
        body,
        grid=(NPAD // BN,),
        in_specs=[
            pl.BlockSpec((BN, 2 * NC), lambda i: (i, 0)),
            pl.BlockSpec((BN, DIN), lambda i: (i, 0)),
            pl.BlockSpec((DIN, DH), lambda i: (0, 0)),
        ],
        out_specs=pl.BlockSpec((BN, DH), lambda i: (i, 0)),
        out_shape=jax.ShapeDtypeStruct((N, DH), jnp.float32),
    )


def _make_tc_mid(N, NPAD, DH, DO, BN):
    def body(hist_ref, aggp_ref, b1_ref, w2_ref, out_ref):
        h = hist_ref[...]
        deg_out = jnp.sum(h[:, :NC], axis=1, keepdims=True)
        deg_in = jnp.sum(h[:, NC:], axis=1, keepdims=True)
        ns = lax.rsqrt(jnp.maximum(deg_out, 1.0))
        nd = lax.rsqrt(jnp.maximum(deg_in, 1.0))
        agg = aggp_ref[0] + aggp_ref[1]
        out1 = jnp.maximum(agg * nd + b1_ref[...], 0.0)
        res = jnp.dot(out1, w2_ref[...],
                      preferred_element_type=jnp.float32) * ns
        out_ref[...] = res.T

    return pl.pallas_call(
        body,
        grid=(NPAD // BN,),
        in_specs=[
            pl.BlockSpec((BN, 2 * NC), lambda i: (i, 0)),
            pl.BlockSpec((NC, BN, DH), lambda i: (0, i, 0)),
            pl.BlockSpec((1, DH), lambda i: (0, 0)),
            pl.BlockSpec((DH, DO), lambda i: (0, 0)),
        ],
        out_specs=pl.BlockSpec((DO, BN), lambda i: (0, i)),
        out_shape=jax.ShapeDtypeStruct((DO, NPAD), jnp.float32),
    )


def _make_tc_final(N, NPAD, DO, BN):
    def body(hist_ref, aggp_ref, b2_ref, out_ref):
        h = hist_ref[...]
        deg_in = jnp.sum(h[:, NC:], axis=1, keepdims=True)
        nd = lax.rsqrt(jnp.maximum(deg_in, 1.0))
        agg = aggp_ref[0, :, 0] + aggp_ref[1, :, 0]
        out_ref[...] = agg.T * nd + b2_ref[...]

    return pl.pallas_call(
        body,
        grid=(NPAD // BN,),
        in_specs=[
            pl.BlockSpec((BN, 2 * NC), lambda i: (i, 0)),
            pl.BlockSpec((NC, DO, 1, BN), lambda i: (0, 0, 0, i)),
            pl.BlockSpec((1, DO), lambda i: (0, 0)),
        ],
        out_specs=pl.BlockSpec((BN, DO), lambda i: (i, 0)),
        out_shape=jax.ShapeDtypeStruct((N, DO), jnp.float32),
    )


def kernel(features, edge_index, W1, b1, W2, b2):
    N, DIN = features.shape
    E = edge_index.shape[1]
    DH = W1.shape[1]
    DO = W2.shape[1]

    KB = 80
    CHB = 25
    CED = 10000
    assert E % (NW * KB * CHB) == 0 and N % NS == 0 and E % (NC * CED) == 0
    CE = E // NW
    NB = CE // KB
    BN = 1024
    NPAD = -(-N // BN) * BN

    edges_deg = edge_index.reshape(2, NW, CE)
    edges_blk = edge_index.reshape(2, NW, NB // CHB, CHB, KB)
    edges_half = edge_index.reshape(2, NC, E // (NC * CED), 1, CED)

    histp = _make_degree_kernel(E, NPAD)(edges_deg)
    hist_t = (histp.reshape(NC, 2, NPAD)
              .transpose(2, 1, 0).reshape(NPAD, 2 * NC))

    h1s = _make_tc_layer1(N, NPAD, DIN, DH, BN)(hist_t, features, W1)
    aggp1 = _make_agg_kernel(NPAD, E, DH, KB, CHB)(edges_blk, h1s)
    h2sT = _make_tc_mid(N, NPAD, DH, DO, BN)(
        hist_t, aggp1, b1.reshape(1, DH), W2)

    aggp2 = _make_agg_cols_kernel(NPAD, E, DO, CED)(
        h2sT.reshape(DO, 1, NPAD), edges_half)

    out = _make_tc_final(N, NPAD, DO, BN)(
        hist_t, aggp2, b2.reshape(1, DO))
    return out


# speedup vs baseline: 1.3824x; 1.0028x over previous
import functools

import jax
import jax.numpy as jnp
from jax import lax
from jax.experimental import pallas as pl
from jax.experimental.pallas import tpu as pltpu
from jax.experimental.pallas import tpu_sc as plsc

NC = 2
NS = 16
NW = NC * NS
L = 16


def _sc_mesh():
    return plsc.VectorSubcoreMesh(core_axis_name="c", subcore_axis_name="s",
                                  num_cores=NC, num_subcores=NS)


def _sc_params():
    return pltpu.CompilerParams(needs_layout_passes=False)


def _make_degree_kernel(E, NPAD):
    CE = E // NW
    CW = 2 * NPAD // NS

    @functools.partial(
        pl.kernel,
        out_type=jax.ShapeDtypeStruct((NC, 1, 2 * NPAD), jnp.float32),
        mesh=_sc_mesh(),
        compiler_params=_sc_params(),
        scratch_types=[
            pltpu.VMEM((2, CE), jnp.int32),
            pltpu.VMEM((2 * NPAD,), jnp.float32),
            pltpu.VMEM((CW,), jnp.float32),
            pltpu.VMEM((CW,), jnp.float32),
            pltpu.VMEM_SHARED((NS, 2 * NPAD), jnp.float32),
        ],
    )
    def deg_k(edges, out, idx_v, hist_v, acc_v, tmp_v, hist_sh):
        c = lax.axis_index("c")
        s = lax.axis_index("s")
        wid = c * NS + s

        zeros = jnp.zeros((L,), jnp.float32)

        def zero_body(i, _):
            hist_v[pl.ds(i * L, L)] = zeros
            return _

        lax.fori_loop(0, 2 * NPAD // L, zero_body, None)

        pltpu.sync_copy(edges.at[0, wid], idx_v.at[0])
        pltpu.sync_copy(edges.at[1, wid], idx_v.at[1])

        ones = jnp.ones((L,), jnp.float32)
        off = jnp.full((L,), NPAD, jnp.int32)

        @plsc.parallel_loop(0, CE // L, unroll=8)
        def _(i):
            sv = idx_v[0, pl.ds(i * L, L)]
            dv = idx_v[1, pl.ds(i * L, L)]
            plsc.addupdate_scatter(hist_v, [sv], ones)
            plsc.addupdate_scatter(hist_v, [dv + off], ones)

        pltpu.sync_copy(hist_v, hist_sh.at[s])
        plsc.subcore_barrier()

        def zacc(i, _):
            acc_v[pl.ds(i * L, L)] = zeros
            return _

        lax.fori_loop(0, CW // L, zacc, None)

        def red_tile(t, _):
            pltpu.sync_copy(hist_sh.at[t, pl.ds(s * CW, CW)], tmp_v)

            @plsc.parallel_loop(0, CW // L, unroll=8)
            def _(i):
                acc_v[pl.ds(i * L, L)] += tmp_v[pl.ds(i * L, L)]
            return _

        lax.fori_loop(0, NS, red_tile, None)
        pltpu.sync_copy(acc_v, out.at[c, 0, pl.ds(s * CW, CW)])

    return deg_k


def _make_agg_kernel(NP, E, D, KB, CHB):
    CE = E // NW
    NB = CE // KB
    NCH = NB // CHB
    assert NB % CHB == 0
    NR = NP // NS
    assert NR % KB == 0 and NR % 8 == 0 and KB % 8 == 0

    @functools.partial(
        pl.kernel,
        out_type=jax.ShapeDtypeStruct((NC, NP, D), jnp.float32),
        mesh=_sc_mesh(),
        compiler_params=_sc_params(),
        scratch_types=[
            pltpu.VMEM((CHB, KB), jnp.int32),
            pltpu.VMEM((CHB, KB), jnp.int32),
            pltpu.VMEM((KB, D), jnp.float32),
            pltpu.VMEM((KB, D), jnp.float32),
            pltpu.VMEM((KB, D), jnp.float32),
            pltpu.VMEM_SHARED((NP, D), jnp.float32),
        ] + [pltpu.SemaphoreType.DMA] * 6,
    )
    def agg_k(edges, table, out, src_v, dst_v, rows0_v, rows1_v, rows2_v,
              agg_sh, gsem0, gsem1, gsem2, ssem0, ssem1, ssem2):
        c = lax.axis_index("c")
        s = lax.axis_index("s")
        wid = c * NS + s

        zeros = jnp.zeros((L,), jnp.float32)
        DL = D // L

        def zero_body(i, _):
            rows0_v[i // DL, pl.ds((i % DL) * L, L)] = zeros
            return _

        lax.fori_loop(0, KB * DL, zero_body, None)

        def zero_dma(k, _):
            pltpu.sync_copy(rows0_v, agg_sh.at[pl.ds(s * NR + k * KB, KB)])
            return _

        lax.fori_loop(0, NR // KB, zero_dma, None)
        plsc.subcore_barrier()

        R = 3
        bufs = (rows0_v, rows1_v, rows2_v)
        gsems = (gsem0, gsem1, gsem2)
        ssems = (ssem0, ssem1, ssem2)

        def gather(j, p):
            return pltpu.make_async_copy(table.at[src_v.at[j]], bufs[p],
                                         gsems[p])

        def scat_wait(j, p):
            return pltpu.make_async_copy(bufs[p], agg_sh.at[dst_v.at[j]],
                                         ssems[p])

        def chunk_body(ch, _):
            pltpu.sync_copy(edges.at[0, wid, ch], src_v)
            pltpu.sync_copy(edges.at[1, wid, ch], dst_v)
            for k in range(R - 1):
                gather(k, k).start()

            def blk_body(j, _):
                for p in range(R):
                    @pl.when(j % R == p)
                    def _():
                        gather(j, p).wait()
                        pltpu.async_copy(bufs[p], agg_sh.at[dst_v.at[j]],
                                         ssems[p], add=True)

                        @pl.when(j + R - 1 < CHB)
                        def _():
                            q = (p + R - 1) % R
                            @pl.when(j > 0)
                            def _():
                                scat_wait(j - 1, q).wait()
                            gather(j + R - 1, q).start()
                return _

            lax.fori_loop(0, CHB, blk_body, None)
            for t in range(CHB - R, CHB):
                scat_wait(t, t % R).wait()
            return _

        lax.fori_loop(0, NCH, chunk_body, None)
        plsc.subcore_barrier()

        pltpu.sync_copy(agg_sh.at[pl.ds(s * NR, NR)],
                        out.at[c, pl.ds(s * NR, NR)])

    return agg_k


def _make_agg_cols_kernel(NP, E, DO, CED):
    EC = E // NC
    NCH = EC // CED
    assert EC % CED == 0 and DO == NS

    @functools.partial(
        pl.kernel,
        out_type=jax.ShapeDtypeStruct((NC, DO, 1, NP), jnp.float32),
        mesh=_sc_mesh(),
        compiler_params=_sc_params(),
        scratch_types=[
            pltpu.VMEM((NP,), jnp.float32),
            pltpu.VMEM((NP,), jnp.float32),
            pltpu.VMEM((2, CED), jnp.int32),
            pltpu.VMEM((2, CED), jnp.int32),
            pltpu.SemaphoreType.DMA,
            pltpu.SemaphoreType.DMA,
        ],
    )
    def aggc_k(tableT, edges, out, col_v, acc_v, idxA_v, idxB_v, semA, semB):
        c = lax.axis_index("c")
        s = lax.axis_index("s")

        zeros = jnp.zeros((L,), jnp.float32)

        def zero_body(i, _):
            acc_v[pl.ds(i * L, L)] = zeros
            return _

        lax.fori_loop(0, NP // L, zero_body, None)

        pltpu.sync_copy(tableT.at[s, 0], col_v)

        ibufs = (idxA_v, idxB_v)
        isems = (semA, semB)

        def idx_load(ch, p):
            def one(r):
                return pltpu.make_async_copy(
                    edges.at[r, c, ch, 0], ibufs[p].at[r], isems[p])
            return one

        idx_load(0, 0)(0).start()
        idx_load(0, 0)(1).start()

        def chunk_body(ch, _):
            for p in range(2):
                @pl.when(ch % 2 == p)
                def _():
                    idx_load(ch, p)(0).wait()
                    idx_load(ch, p)(1).wait()

                    @pl.when(ch < NCH - 1)
                    def _():
                        idx_load(ch + 1, 1 - p)(0).start()
                        idx_load(ch + 1, 1 - p)(1).start()

                    @plsc.parallel_loop(0, CED // L, unroll=16)
                    def _(k):
                        sv = ibufs[p][0, pl.ds(k * L, L)]
                        dv = ibufs[p][1, pl.ds(k * L, L)]
                        vals = plsc.load_gather(col_v, [sv])
                        plsc.addupdate_scatter(acc_v, [dv], vals)
            return _

        lax.fori_loop(0, NCH, chunk_body, None)

        pltpu.sync_copy(acc_v, out.at[c, s, 0])

    return aggc_k


def _make_tc_layer1(N, NPAD, DIN, DH, BN):
    def body(hist_ref, feat_ref, w_ref, out_ref):
        h = hist_ref[...]
        deg_out = jnp.sum(h[:, :NC], axis=1, keepdims=True)
        ns = lax.rsqrt(jnp.maximum(deg_out, 1.0))
        x = feat_ref[...] * ns
        out_ref[...] = jnp.dot(x, w_ref[...], preferred_element_type=jnp.float32)

    return pl.pallas_call(
        body,
        grid=(NPAD // BN,),
        in_specs=[
            pl.BlockSpec((BN, 2 * NC), lambda i: (i, 0)),
            pl.BlockSpec((BN, DIN), lambda i: (i, 0)),
            pl.BlockSpec((DIN, DH), lambda i: (0, 0)),
        ],
        out_specs=pl.BlockSpec((BN, DH), lambda i: (i, 0)),
        out_shape=jax.ShapeDtypeStruct((N, DH), jnp.float32),
    )


def _make_tc_mid(N, NPAD, DH, DO, BN):
    def body(hist_ref, aggp_ref, b1_ref, w2_ref, out_ref):
        h = hist_ref[...]
        deg_out = jnp.sum(h[:, :NC], axis=1, keepdims=True)
        deg_in = jnp.sum(h[:, NC:], axis=1, keepdims=True)
        ns = lax.rsqrt(jnp.maximum(deg_out, 1.0))
        nd = lax.rsqrt(jnp.maximum(deg_in, 1.0))
        agg = aggp_ref[0] + aggp_ref[1]
        out1 = jnp.maximum(agg * nd + b1_ref[...], 0.0)
        res = jnp.dot(out1, w2_ref[...],
                      preferred_element_type=jnp.float32) * ns
        out_ref[...] = res.T

    return pl.pallas_call(
        body,
        grid=(NPAD // BN,),
        in_specs=[
            pl.BlockSpec((BN, 2 * NC), lambda i: (i, 0)),
            pl.BlockSpec((NC, BN, DH), lambda i: (0, i, 0)),
            pl.BlockSpec((1, DH), lambda i: (0, 0)),
            pl.BlockSpec((DH, DO), lambda i: (0, 0)),
        ],
        out_specs=pl.BlockSpec((DO, BN), lambda i: (0, i)),
        out_shape=jax.ShapeDtypeStruct((DO, NPAD), jnp.float32),
    )


def _make_tc_final(N, NPAD, DO, BN):
    def body(hist_ref, aggp_ref, b2_ref, out_ref):
        h = hist_ref[...]
        deg_in = jnp.sum(h[:, NC:], axis=1, keepdims=True)
        nd = lax.rsqrt(jnp.maximum(deg_in, 1.0))
        agg = aggp_ref[0, :, 0] + aggp_ref[1, :, 0]
        out_ref[...] = agg.T * nd + b2_ref[...]

    return pl.pallas_call(
        body,
        grid=(NPAD // BN,),
        in_specs=[
            pl.BlockSpec((BN, 2 * NC), lambda i: (i, 0)),
            pl.BlockSpec((NC, DO, 1, BN), lambda i: (0, 0, 0, i)),
            pl.BlockSpec((1, DO), lambda i: (0, 0)),
        ],
        out_specs=pl.BlockSpec((BN, DO), lambda i: (i, 0)),
        out_shape=jax.ShapeDtypeStruct((N, DO), jnp.float32),
    )


def kernel(features, edge_index, W1, b1, W2, b2):
    N, DIN = features.shape
    E = edge_index.shape[1]
    DH = W1.shape[1]
    DO = W2.shape[1]

    KB = 80
    CHB = 25
    CED = 10000
    assert E % (NW * KB * CHB) == 0 and N % NS == 0 and E % (NC * CED) == 0
    CE = E // NW
    NB = CE // KB
    BN = 1024
    NPAD = -(-N // BN) * BN

    edges_deg = edge_index.reshape(2, NW, CE)
    edges_blk = edge_index.reshape(2, NW, NB // CHB, CHB, KB)
    edges_half = edge_index.reshape(2, NC, E // (NC * CED), 1, CED)

    histp = _make_degree_kernel(E, NPAD)(edges_deg)
    hist_t = (histp.reshape(NC, 2, NPAD)
              .transpose(2, 1, 0).reshape(NPAD, 2 * NC))

    h1s = _make_tc_layer1(N, NPAD, DIN, DH, BN)(hist_t, features, W1)
    aggp1 = _make_agg_kernel(NPAD, E, DH, KB, CHB)(edges_blk, h1s)
    h2sT = _make_tc_mid(N, NPAD, DH, DO, BN)(
        hist_t, aggp1, b1.reshape(1, DH), W2)

    aggp2 = _make_agg_cols_kernel(NPAD, E, DO, CED)(
        h2sT.reshape(DO, 1, NPAD), edges_half)

    out = _make_tc_final(N, NPAD, DO, BN)(
        hist_t, aggp2, b2.reshape(1, DO))
    return out
